# Initial kernel scaffold; baseline (speedup 1.0000x reference)
import jax
import jax.numpy as jnp
from jax.experimental import pallas as pl


def kernel(idx, target, table):
    raise NotImplementedError("write your pallas kernel here")


# speedup vs baseline: 1.5438x; 1.5438x over previous
import functools

import jax
import jax.numpy as jnp
from jax import lax
from jax.experimental import pallas as pl
from jax.experimental.pallas import tpu as pltpu
from jax.experimental.pallas import tpu_sc as plsc

V = 1000
N = 51200
VA = 896
NC, NS, L = 2, 16, 16
NW = NC * NS
RPT = N // NW
G = 64
NCHUNK = RPT // G
GT = 512
NBT = N // GT
TW = V - VA
TSTART = VA


def _logz_body(tab_ref, out_ref):
    x = tab_ref[:]
    m = jnp.max(x, axis=1, keepdims=True)
    s = jnp.sum(jnp.exp(x - m), axis=1, keepdims=True)
    out_ref[:] = m + jnp.log(s)


def _tc_logz(table):
    return pl.pallas_call(
        _logz_body,
        out_shape=jax.ShapeDtypeStruct((V, 1), jnp.float32),
    )(table)


def _sc_gather(table_a, table_flat, idx_flat, tgt_flat, logz):
    mesh = plsc.VectorSubcoreMesh(core_axis_name="c", subcore_axis_name="s")

    @functools.partial(
        pl.kernel,
        out_type=(
            jax.ShapeDtypeStruct((N, V), jnp.float32),
            jax.ShapeDtypeStruct((NW, L), jnp.float32),
        ),
        mesh=mesh,
        scratch_types=[
            pltpu.VMEM((RPT,), jnp.int32),
            pltpu.VMEM((RPT,), jnp.int32),
            pltpu.VMEM((RPT,), jnp.int32),
            pltpu.VMEM((RPT,), jnp.float32),
            pltpu.VMEM((RPT,), jnp.float32),
            pltpu.VMEM((G, VA), jnp.float32),
            pltpu.VMEM((L,), jnp.float32),
            pltpu.SemaphoreType.DMA,
        ],
    )
    def k(taba_hbm, tabflat_hbm, idx_hbm, tgt_hbm, logz_hbm, out_hbm,
          part_hbm, idx_v, tgt_v, fidx_v, picked_v, lzp_v, rows_v, acc_v,
          sem):
        wid = lax.axis_index("s") * NC + lax.axis_index("c")
        base = pl.multiple_of(wid * RPT, RPT)
        pltpu.sync_copy(idx_hbm.at[pl.ds(base, RPT)], idx_v)
        pltpu.sync_copy(tgt_hbm.at[pl.ds(base, RPT)], tgt_v)

        def mkflat(j, _):
            o = pl.multiple_of(j * L, L)
            fidx_v[pl.ds(o, L)] = idx_v[pl.ds(o, L)] * V + tgt_v[pl.ds(o, L)]
            return 0
        lax.fori_loop(0, RPT // L, mkflat, 0)

        pltpu.async_copy(tabflat_hbm.at[fidx_v], picked_v, sem).wait()
        pltpu.async_copy(logz_hbm.at[idx_v], lzp_v, sem).wait()

        def accum(j, acc):
            o = pl.multiple_of(j * L, L)
            return acc + (lzp_v[pl.ds(o, L)] - picked_v[pl.ds(o, L)])
        acc = lax.fori_loop(0, RPT // L, accum, jnp.zeros((L,), jnp.float32))
        acc_v[...] = acc
        pltpu.sync_copy(acc_v, part_hbm.at[wid])

        def chunk(g, _):
            off = pl.multiple_of(g * G, G)
            pltpu.async_copy(taba_hbm.at[idx_v.at[pl.ds(off, G)]],
                             rows_v, sem).wait()
            pltpu.sync_copy(rows_v,
                            out_hbm.at[pl.ds(base + off, G), pl.ds(0, VA)])
            return 0
        lax.fori_loop(0, NCHUNK, chunk, 0)

    return k(table_a, table_flat, idx_flat, tgt_flat, logz)


def _tail_body(idx_ref, tail_ref, part_ref, alias_ref, out_ref, loss_ref,
               acc_ref, sem):
    del alias_ref
    i = pl.program_id(0)
    ids = idx_ref[0, 0, :]
    oh = (ids[:, None] == lax.broadcasted_iota(jnp.int32, (GT, V), 1))
    acc_ref[...] = jnp.dot(oh.astype(jnp.float32), tail_ref[...],
                           preferred_element_type=jnp.float32)
    pltpu.async_copy(
        acc_ref, out_ref.at[pl.ds(i * GT, GT), pl.ds(TSTART, TW)], sem
    ).wait()

    @pl.when(i == 0)
    def _():
        loss_ref[0, 0] = jnp.sum(part_ref[...]) * (1.0 / N)


def _tc_tail(idx3, tail, partials, sc_out):
    return pl.pallas_call(
        _tail_body,
        grid=(NBT,),
        in_specs=[
            pl.BlockSpec((1, 1, GT), lambda i: (i, 0, 0)),
            pl.BlockSpec((V, TW), lambda i: (0, 0)),
            pl.BlockSpec((NW, L), lambda i: (0, 0)),
            pl.BlockSpec(memory_space=pl.ANY),
        ],
        out_specs=[
            pl.BlockSpec(memory_space=pl.ANY),
            pl.BlockSpec((1, 1), lambda i: (0, 0),
                         memory_space=pltpu.SMEM),
        ],
        out_shape=[
            jax.ShapeDtypeStruct((N, V), jnp.float32),
            jax.ShapeDtypeStruct((1, 1), jnp.float32),
        ],
        scratch_shapes=[
            pltpu.VMEM((GT, TW), jnp.float32),
            pltpu.SemaphoreType.DMA,
        ],
        input_output_aliases={3: 0},
    )(idx3, tail, partials, sc_out)


def kernel(idx, target, table):
    idx_flat = idx.reshape(N)
    tgt_flat = target.reshape(N)
    logz = _tc_logz(table).reshape(V)
    table_a = table[:, :VA]
    tail = table[:, TSTART:]
    sc_out, partials = _sc_gather(table_a, table.reshape(V * V), idx_flat,
                                  tgt_flat, logz)
    logits2, loss = _tc_tail(idx_flat.reshape(NBT, 1, GT), tail, partials,
                             sc_out)
    return (logits2, loss[0, 0])


# speedup vs baseline: 1.5749x; 1.0201x over previous
import functools

import jax
import jax.numpy as jnp
from jax import lax
from jax.experimental import pallas as pl
from jax.experimental.pallas import tpu as pltpu
from jax.experimental.pallas import tpu_sc as plsc

V = 1000
N = 51200
VA = 896
NC, NS, L = 2, 16, 16
NW = NC * NS
RPT = N // NW
G = 32
NCHUNK = RPT // G
GT = 512
NBT = N // GT
TW = V - VA
TSTART = VA


def _logz_body(tab_ref, out_ref):
    x = tab_ref[:]
    m = jnp.max(x, axis=1, keepdims=True)
    s = jnp.sum(jnp.exp(x - m), axis=1, keepdims=True)
    out_ref[:] = m + jnp.log(s)


def _tc_logz(table):
    return pl.pallas_call(
        _logz_body,
        out_shape=jax.ShapeDtypeStruct((V, 1), jnp.float32),
    )(table)


def _sc_gather(table_a, table_flat, idx_flat, tgt_flat, logz):
    mesh = plsc.VectorSubcoreMesh(core_axis_name="c", subcore_axis_name="s")

    @functools.partial(
        pl.kernel,
        out_type=(
            jax.ShapeDtypeStruct((N, V), jnp.float32),
            jax.ShapeDtypeStruct((NW, L), jnp.float32),
        ),
        mesh=mesh,
        scratch_types=[
            pltpu.VMEM((RPT,), jnp.int32),
            pltpu.VMEM((RPT,), jnp.int32),
            pltpu.VMEM((RPT,), jnp.int32),
            pltpu.VMEM((RPT,), jnp.float32),
            pltpu.VMEM((RPT,), jnp.float32),
            pltpu.VMEM((G, VA), jnp.float32),
            pltpu.VMEM((G, VA), jnp.float32),
            pltpu.VMEM((L,), jnp.float32),
            pltpu.SemaphoreType.DMA,
            pltpu.SemaphoreType.DMA,
            pltpu.SemaphoreType.DMA,
            pltpu.SemaphoreType.DMA,
            pltpu.SemaphoreType.DMA,
        ],
    )
    def k(taba_hbm, tabflat_hbm, idx_hbm, tgt_hbm, logz_hbm, out_hbm,
          part_hbm, idx_v, tgt_v, fidx_v, picked_v, lzp_v, rows0_v, rows1_v,
          acc_v, psem, sin0, sin1, sout0, sout1):
        wid = lax.axis_index("s") * NC + lax.axis_index("c")
        base = pl.multiple_of(wid * RPT, RPT)
        rows = (rows0_v, rows1_v)
        sin = (sin0, sin1)
        sout = (sout0, sout1)
        pltpu.sync_copy(idx_hbm.at[pl.ds(base, RPT)], idx_v)
        pltpu.sync_copy(tgt_hbm.at[pl.ds(base, RPT)], tgt_v)

        def start_in(g, b):
            off = pl.multiple_of(g * G, G)
            pltpu.async_copy(taba_hbm.at[idx_v.at[pl.ds(off, G)]],
                             rows[b], sin[b])

        def wait_in(b):
            pltpu.make_async_copy(taba_hbm.at[idx_v.at[pl.ds(0, G)]],
                                  rows[b], sin[b]).wait()

        def start_out(g, b):
            off = pl.multiple_of(g * G, G)
            pltpu.async_copy(rows[b],
                             out_hbm.at[pl.ds(base + off, G), pl.ds(0, VA)],
                             sout[b])

        def wait_out(b):
            pltpu.make_async_copy(rows[b],
                                  out_hbm.at[pl.ds(base, G), pl.ds(0, VA)],
                                  sout[b]).wait()

        start_in(0, 0)
        start_in(1, 1)

        def mkflat(j, _):
            o = pl.multiple_of(j * L, L)
            fidx_v[pl.ds(o, L)] = idx_v[pl.ds(o, L)] * V + tgt_v[pl.ds(o, L)]
            return 0
        lax.fori_loop(0, RPT // L, mkflat, 0)

        pltpu.async_copy(tabflat_hbm.at[fidx_v], picked_v, psem)
        pltpu.async_copy(logz_hbm.at[idx_v], lzp_v, psem)
        pltpu.make_async_copy(tabflat_hbm.at[fidx_v], picked_v, psem).wait()
        pltpu.make_async_copy(logz_hbm.at[idx_v], lzp_v, psem).wait()

        def accum(j, acc):
            o = pl.multiple_of(j * L, L)
            return acc + (lzp_v[pl.ds(o, L)] - picked_v[pl.ds(o, L)])
        acc = lax.fori_loop(0, RPT // L, accum, jnp.zeros((L,), jnp.float32))
        acc_v[...] = acc
        pltpu.sync_copy(acc_v, part_hbm.at[wid])

        def step(i, _):
            t = i * 2
            for b in range(2):
                g = t + b
                wait_in(b)
                start_out(g, b)

            @pl.when(t + 2 < NCHUNK)
            def _():
                wait_out(0)
                start_in(t + 2, 0)

            @pl.when(t + 3 < NCHUNK)
            def _():
                wait_out(1)
                start_in(t + 3, 1)
            return 0
        lax.fori_loop(0, NCHUNK // 2, step, 0)
        wait_out(0)
        wait_out(1)

    return k(table_a, table_flat, idx_flat, tgt_flat, logz)


def _tail_body(idx_ref, tail_ref, part_ref, alias_ref, out_ref, loss_ref,
               acc_ref, sem):
    del alias_ref
    i = pl.program_id(0)
    ids = idx_ref[0, 0, :]
    oh = (ids[:, None] == lax.broadcasted_iota(jnp.int32, (GT, V), 1))
    acc_ref[...] = jnp.dot(oh.astype(jnp.float32), tail_ref[...],
                           preferred_element_type=jnp.float32)
    pltpu.async_copy(
        acc_ref, out_ref.at[pl.ds(i * GT, GT), pl.ds(TSTART, TW)], sem
    ).wait()

    @pl.when(i == 0)
    def _():
        loss_ref[0, 0] = jnp.sum(part_ref[...]) * (1.0 / N)


def _tc_tail(idx3, tail, partials, sc_out):
    return pl.pallas_call(
        _tail_body,
        grid=(NBT,),
        in_specs=[
            pl.BlockSpec((1, 1, GT), lambda i: (i, 0, 0)),
            pl.BlockSpec((V, TW), lambda i: (0, 0)),
            pl.BlockSpec((NW, L), lambda i: (0, 0)),
            pl.BlockSpec(memory_space=pl.ANY),
        ],
        out_specs=[
            pl.BlockSpec(memory_space=pl.ANY),
            pl.BlockSpec((1, 1), lambda i: (0, 0),
                         memory_space=pltpu.SMEM),
        ],
        out_shape=[
            jax.ShapeDtypeStruct((N, V), jnp.float32),
            jax.ShapeDtypeStruct((1, 1), jnp.float32),
        ],
        scratch_shapes=[
            pltpu.VMEM((GT, TW), jnp.float32),
            pltpu.SemaphoreType.DMA,
        ],
        input_output_aliases={3: 0},
    )(idx3, tail, partials, sc_out)


def kernel(idx, target, table):
    idx_flat = idx.reshape(N)
    tgt_flat = target.reshape(N)
    logz = _tc_logz(table).reshape(V)
    table_a = table[:, :VA]
    tail = table[:, TSTART:]
    sc_out, partials = _sc_gather(table_a, table.reshape(V * V), idx_flat,
                                  tgt_flat, logz)
    logits2, loss = _tc_tail(idx_flat.reshape(NBT, 1, GT), tail, partials,
                             sc_out)
    return (logits2, loss[0, 0])


# speedup vs baseline: 2.2450x; 1.4254x over previous
import functools

import jax
import jax.numpy as jnp
from jax import lax
from jax.experimental import pallas as pl
from jax.experimental.pallas import tpu as pltpu
from jax.experimental.pallas import tpu_sc as plsc

V = 1000
N = 51200
VA = 896
NC, NS, L = 2, 16, 16
NW = NC * NS
RPT = N // NW
G = 32
NCHUNK = RPT // G
GT = 512
NBT = N // GT
TW = V - VA
TSTART = VA


def _logz_body(tab_ref, out_ref):
    x = tab_ref[:]
    m = jnp.max(x, axis=1, keepdims=True)
    s = jnp.sum(jnp.exp(x - m), axis=1, keepdims=True)
    out_ref[:] = m + jnp.log(s)


def _tc_logz(table):
    return pl.pallas_call(
        _logz_body,
        out_shape=jax.ShapeDtypeStruct((V, 1), jnp.float32),
    )(table)


def _sc_gather(table_a, table_flat, idx_flat, tgt_flat, logz):
    mesh = plsc.VectorSubcoreMesh(core_axis_name="c", subcore_axis_name="s")

    @functools.partial(
        pl.kernel,
        out_type=(
            jax.ShapeDtypeStruct((N, VA), jnp.float32),
            jax.ShapeDtypeStruct((NW, L), jnp.float32),
        ),
        mesh=mesh,
        scratch_types=[
            pltpu.VMEM((RPT,), jnp.int32),
            pltpu.VMEM((RPT,), jnp.int32),
            pltpu.VMEM((RPT,), jnp.int32),
            pltpu.VMEM((RPT,), jnp.float32),
            pltpu.VMEM((RPT,), jnp.float32),
            pltpu.VMEM((G, VA), jnp.float32),
            pltpu.VMEM((G, VA), jnp.float32),
            pltpu.VMEM((L,), jnp.float32),
            pltpu.SemaphoreType.DMA,
            pltpu.SemaphoreType.DMA,
            pltpu.SemaphoreType.DMA,
            pltpu.SemaphoreType.DMA,
            pltpu.SemaphoreType.DMA,
        ],
    )
    def k(taba_hbm, tabflat_hbm, idx_hbm, tgt_hbm, logz_hbm, out_hbm,
          part_hbm, idx_v, tgt_v, fidx_v, picked_v, lzp_v, rows0_v, rows1_v,
          acc_v, psem, sin0, sin1, sout0, sout1):
        wid = lax.axis_index("s") * NC + lax.axis_index("c")
        base = pl.multiple_of(wid * RPT, RPT)
        rows = (rows0_v, rows1_v)
        sin = (sin0, sin1)
        sout = (sout0, sout1)
        pltpu.sync_copy(idx_hbm.at[pl.ds(base, RPT)], idx_v)
        pltpu.sync_copy(tgt_hbm.at[pl.ds(base, RPT)], tgt_v)

        def start_in(g, b):
            off = pl.multiple_of(g * G, G)
            pltpu.async_copy(taba_hbm.at[idx_v.at[pl.ds(off, G)]],
                             rows[b], sin[b])

        def wait_in(b):
            pltpu.make_async_copy(taba_hbm.at[idx_v.at[pl.ds(0, G)]],
                                  rows[b], sin[b]).wait()

        def start_out(g, b):
            off = pl.multiple_of(g * G, G)
            pltpu.async_copy(rows[b],
                             out_hbm.at[pl.ds(base + off, G), pl.ds(0, VA)],
                             sout[b])

        def wait_out(b):
            pltpu.make_async_copy(rows[b],
                                  out_hbm.at[pl.ds(base, G), pl.ds(0, VA)],
                                  sout[b]).wait()

        start_in(0, 0)
        start_in(1, 1)

        def mkflat(j, _):
            o = pl.multiple_of(j * L, L)
            fidx_v[pl.ds(o, L)] = idx_v[pl.ds(o, L)] * V + tgt_v[pl.ds(o, L)]
            return 0
        lax.fori_loop(0, RPT // L, mkflat, 0)

        pltpu.async_copy(tabflat_hbm.at[fidx_v], picked_v, psem)
        pltpu.async_copy(logz_hbm.at[idx_v], lzp_v, psem)
        pltpu.make_async_copy(tabflat_hbm.at[fidx_v], picked_v, psem).wait()
        pltpu.make_async_copy(logz_hbm.at[idx_v], lzp_v, psem).wait()

        def accum(j, acc):
            o = pl.multiple_of(j * L, L)
            return acc + (lzp_v[pl.ds(o, L)] - picked_v[pl.ds(o, L)])
        acc = lax.fori_loop(0, RPT // L, accum, jnp.zeros((L,), jnp.float32))
        acc_v[...] = acc
        pltpu.sync_copy(acc_v, part_hbm.at[wid])

        def step(i, _):
            t = i * 2
            for b in range(2):
                g = t + b
                wait_in(b)
                start_out(g, b)

            @pl.when(t + 2 < NCHUNK)
            def _():
                wait_out(0)
                start_in(t + 2, 0)

            @pl.when(t + 3 < NCHUNK)
            def _():
                wait_out(1)
                start_in(t + 3, 1)
            return 0
        lax.fori_loop(0, NCHUNK // 2, step, 0)
        wait_out(0)
        wait_out(1)

    return k(table_a, table_flat, idx_flat, tgt_flat, logz)


def _relayout_body(idx_ref, tailT_ref, part_ref, sc_ref, out_ref, loss_ref):
    i = pl.program_id(0)
    ids = idx_ref[0, 0, :]
    out_ref[0:VA, :] = jnp.transpose(sc_ref[...])
    ohT = (lax.broadcasted_iota(jnp.int32, (V, GT), 0)
           == ids[None, :]).astype(jnp.float32)
    out_ref[VA:V, :] = jnp.dot(tailT_ref[...], ohT,
                               preferred_element_type=jnp.float32)

    @pl.when(i == 0)
    def _():
        loss_ref[0, 0] = jnp.sum(part_ref[...]) * (1.0 / N)


def _tc_relayout(idx3, tailT, partials, sc_out):
    return pl.pallas_call(
        _relayout_body,
        grid=(NBT,),
        in_specs=[
            pl.BlockSpec((1, 1, GT), lambda i: (i, 0, 0)),
            pl.BlockSpec((TW, V), lambda i: (0, 0)),
            pl.BlockSpec((NW, L), lambda i: (0, 0)),
            pl.BlockSpec((GT, VA), lambda i: (i, 0)),
        ],
        out_specs=[
            pl.BlockSpec((V, GT), lambda i: (0, i)),
            pl.BlockSpec((1, 1), lambda i: (0, 0),
                         memory_space=pltpu.SMEM),
        ],
        out_shape=[
            jax.ShapeDtypeStruct((V, N), jnp.float32),
            jax.ShapeDtypeStruct((1, 1), jnp.float32),
        ],
    )(idx3, tailT, partials, sc_out)


def kernel(idx, target, table):
    idx_flat = idx.reshape(N)
    tgt_flat = target.reshape(N)
    logz = _tc_logz(table).reshape(V)
    table_a = table[:, :VA]
    tailT = table[:, TSTART:].T
    sc_out, partials = _sc_gather(table_a, table.reshape(V * V), idx_flat,
                                  tgt_flat, logz)
    outT, loss = _tc_relayout(idx_flat.reshape(NBT, 1, GT), tailT, partials,
                              sc_out)
    return (jnp.transpose(outT), loss[0, 0])


# speedup vs baseline: 2.2838x; 1.0173x over previous
import functools

import jax
import jax.numpy as jnp
from jax import lax
from jax.experimental import pallas as pl
from jax.experimental.pallas import tpu as pltpu
from jax.experimental.pallas import tpu_sc as plsc

V = 1000
N = 51200
NH = N // 2
VA = 896
NC, NS, L = 2, 16, 16
NW = NC * NS
RPT = NH // NW
G = 16
NCHUNK = RPT // G
GT = 512
NBH = NH // GT
TW = V - VA
TSTART = VA


def _logz_body(tab_ref, out_ref):
    x = tab_ref[:]
    m = jnp.max(x, axis=1, keepdims=True)
    s = jnp.sum(jnp.exp(x - m), axis=1, keepdims=True)
    out_ref[:] = m + jnp.log(s)


def _tc_logz(table):
    return pl.pallas_call(
        _logz_body,
        out_shape=jax.ShapeDtypeStruct((V, 1), jnp.float32),
    )(table)


def _sc_gather(table_a, table_flat, idx_h, tgt_h, logz):
    mesh = plsc.VectorSubcoreMesh(core_axis_name="c", subcore_axis_name="s")

    @functools.partial(
        pl.kernel,
        out_type=(
            jax.ShapeDtypeStruct((NH, VA), jnp.float32),
            jax.ShapeDtypeStruct((NW, L), jnp.float32),
        ),
        mesh=mesh,
        scratch_types=[
            pltpu.VMEM((RPT,), jnp.int32),
            pltpu.VMEM((RPT,), jnp.int32),
            pltpu.VMEM((RPT,), jnp.int32),
            pltpu.VMEM((RPT,), jnp.float32),
            pltpu.VMEM((RPT,), jnp.float32),
            pltpu.VMEM((G, VA), jnp.float32),
            pltpu.VMEM((G, VA), jnp.float32),
            pltpu.VMEM((L,), jnp.float32),
            pltpu.SemaphoreType.DMA,
            pltpu.SemaphoreType.DMA,
            pltpu.SemaphoreType.DMA,
            pltpu.SemaphoreType.DMA,
            pltpu.SemaphoreType.DMA,
        ],
    )
    def k(taba_hbm, tabflat_hbm, idx_hbm, tgt_hbm, logz_hbm, out_hbm,
          part_hbm, idx_v, tgt_v, fidx_v, picked_v, lzp_v, rows0_v, rows1_v,
          acc_v, psem, sin0, sin1, sout0, sout1):
        wid = lax.axis_index("s") * NC + lax.axis_index("c")
        base = pl.multiple_of(wid * RPT, RPT)
        rows = (rows0_v, rows1_v)
        sin = (sin0, sin1)
        sout = (sout0, sout1)
        pltpu.sync_copy(idx_hbm.at[pl.ds(base, RPT)], idx_v)
        pltpu.sync_copy(tgt_hbm.at[pl.ds(base, RPT)], tgt_v)

        def start_in(g, b):
            off = pl.multiple_of(g * G, G)
            pltpu.async_copy(taba_hbm.at[idx_v.at[pl.ds(off, G)]],
                             rows[b], sin[b])

        def wait_in(b):
            pltpu.make_async_copy(taba_hbm.at[idx_v.at[pl.ds(0, G)]],
                                  rows[b], sin[b]).wait()

        def start_out(g, b):
            off = pl.multiple_of(g * G, G)
            pltpu.async_copy(rows[b],
                             out_hbm.at[pl.ds(base + off, G), pl.ds(0, VA)],
                             sout[b])

        def wait_out(b):
            pltpu.make_async_copy(rows[b],
                                  out_hbm.at[pl.ds(base, G), pl.ds(0, VA)],
                                  sout[b]).wait()

        start_in(0, 0)
        start_in(1, 1)

        def mkflat(j, _):
            o = pl.multiple_of(j * L, L)
            fidx_v[pl.ds(o, L)] = idx_v[pl.ds(o, L)] * V + tgt_v[pl.ds(o, L)]
            return 0
        lax.fori_loop(0, RPT // L, mkflat, 0)

        pltpu.async_copy(tabflat_hbm.at[fidx_v], picked_v, psem)
        pltpu.async_copy(logz_hbm.at[idx_v], lzp_v, psem)
        pltpu.make_async_copy(tabflat_hbm.at[fidx_v], picked_v, psem).wait()
        pltpu.make_async_copy(logz_hbm.at[idx_v], lzp_v, psem).wait()

        def accum(j, acc):
            o = pl.multiple_of(j * L, L)
            return acc + (lzp_v[pl.ds(o, L)] - picked_v[pl.ds(o, L)])
        acc = lax.fori_loop(0, RPT // L, accum, jnp.zeros((L,), jnp.float32))
        acc_v[...] = acc
        pltpu.sync_copy(acc_v, part_hbm.at[wid])

        def step(i, _):
            t = i * 2
            for b in range(2):
                g = t + b
                wait_in(b)
                start_out(g, b)

            @pl.when(t + 2 < NCHUNK)
            def _():
                wait_out(0)
                start_in(t + 2, 0)

            @pl.when(t + 3 < NCHUNK)
            def _():
                wait_out(1)
                start_in(t + 3, 1)
            return 0
        lax.fori_loop(0, NCHUNK // 2, step, 0)
        wait_out(0)
        wait_out(1)

    return k(table_a, table_flat, idx_h, tgt_h, logz)


def _oh_tail(idx_ref, tailT_ref):
    ids = idx_ref[0, 0, :]
    ohT = (lax.broadcasted_iota(jnp.int32, (V, GT), 0)
           == ids[None, :]).astype(jnp.float32)
    return jnp.dot(tailT_ref[...], ohT, preferred_element_type=jnp.float32)


def _relayout_a_body(idx_ref, tailT_ref, sc_ref, out_ref):
    out_ref[0:VA, :] = jnp.transpose(sc_ref[...])
    out_ref[VA:V, :] = _oh_tail(idx_ref, tailT_ref)


def _relayout_b_body(idx_ref, tailT_ref, pa_ref, pb_ref, sc_ref, alias_ref,
                     out_ref, loss_ref):
    del alias_ref
    out_ref[0:VA, :] = jnp.transpose(sc_ref[...])
    out_ref[VA:V, :] = _oh_tail(idx_ref, tailT_ref)

    @pl.when(pl.program_id(0) == 0)
    def _():
        loss_ref[0, 0] = (jnp.sum(pa_ref[...]) + jnp.sum(pb_ref[...])) \
            * (1.0 / N)


def _tc_relayout_a(idx3, tailT, sc_a):
    return pl.pallas_call(
        _relayout_a_body,
        grid=(NBH,),
        in_specs=[
            pl.BlockSpec((1, 1, GT), lambda i: (i, 0, 0)),
            pl.BlockSpec((TW, V), lambda i: (0, 0)),
            pl.BlockSpec((GT, VA), lambda i: (i, 0)),
        ],
        out_specs=pl.BlockSpec((V, GT), lambda i: (0, i)),
        out_shape=jax.ShapeDtypeStruct((V, N), jnp.float32),
    )(idx3, tailT, sc_a)


def _tc_relayout_b(idx3, tailT, pa, pb, sc_b, outT):
    return pl.pallas_call(
        _relayout_b_body,
        grid=(NBH,),
        in_specs=[
            pl.BlockSpec((1, 1, GT), lambda i: (i, 0, 0)),
            pl.BlockSpec((TW, V), lambda i: (0, 0)),
            pl.BlockSpec((NW, L), lambda i: (0, 0)),
            pl.BlockSpec((NW, L), lambda i: (0, 0)),
            pl.BlockSpec((GT, VA), lambda i: (i, 0)),
            pl.BlockSpec(memory_space=pl.ANY),
        ],
        out_specs=[
            pl.BlockSpec((V, GT), lambda i: (0, i + NBH)),
            pl.BlockSpec((1, 1), lambda i: (0, 0),
                         memory_space=pltpu.SMEM),
        ],
        out_shape=[
            jax.ShapeDtypeStruct((V, N), jnp.float32),
            jax.ShapeDtypeStruct((1, 1), jnp.float32),
        ],
        input_output_aliases={5: 0},
    )(idx3, tailT, pa, pb, sc_b, outT)


def kernel(idx, target, table):
    idx_flat = idx.reshape(N)
    tgt_flat = target.reshape(N)
    logz = _tc_logz(table).reshape(V)
    table_a = table[:, :VA]
    table_flat = table.reshape(V * V)
    tailT = table[:, TSTART:].T
    idx_a, idx_b = idx_flat[:NH], idx_flat[NH:]
    tgt_a, tgt_b = tgt_flat[:NH], tgt_flat[NH:]
    sc_a, pa = _sc_gather(table_a, table_flat, idx_a, tgt_a, logz)
    sc_b, pb = _sc_gather(table_a, table_flat, idx_b, tgt_b, logz)
    outT_a = _tc_relayout_a(idx_a.reshape(NBH, 1, GT), tailT, sc_a)
    outT, loss = _tc_relayout_b(idx_b.reshape(NBH, 1, GT), tailT, pa, pb,
                                sc_b, outT_a)
    return (jnp.transpose(outT), loss[0, 0])


# speedup vs baseline: 2.3543x; 1.0309x over previous
import functools

import jax
import jax.numpy as jnp
from jax import lax
from jax.experimental import pallas as pl
from jax.experimental.pallas import tpu as pltpu
from jax.experimental.pallas import tpu_sc as plsc

V = 1000
N = 51200
NH = N // 2
VA = 896
NC, NS, L = 2, 16, 16
NW = NC * NS
RPT = NH // NW
G = 16
NCHUNK = RPT // G
GT = 1024
NBH = NH // GT
TW = V - VA
TSTART = VA


def _logz_body(tab_ref, out_ref):
    x = tab_ref[:]
    m = jnp.max(x, axis=1, keepdims=True)
    s = jnp.sum(jnp.exp(x - m), axis=1, keepdims=True)
    out_ref[:] = m + jnp.log(s)


def _tc_logz(table):
    return pl.pallas_call(
        _logz_body,
        out_shape=jax.ShapeDtypeStruct((V, 1), jnp.float32),
    )(table)


def _sc_gather(table_a, table_flat, idx_h, tgt_h, logz):
    mesh = plsc.VectorSubcoreMesh(core_axis_name="c", subcore_axis_name="s")

    @functools.partial(
        pl.kernel,
        out_type=(
            jax.ShapeDtypeStruct((NH, VA), jnp.float32),
            jax.ShapeDtypeStruct((NW, L), jnp.float32),
        ),
        mesh=mesh,
        scratch_types=[
            pltpu.VMEM((RPT,), jnp.int32),
            pltpu.VMEM((RPT,), jnp.int32),
            pltpu.VMEM((RPT,), jnp.int32),
            pltpu.VMEM((RPT,), jnp.float32),
            pltpu.VMEM((RPT,), jnp.float32),
            pltpu.VMEM((G, VA), jnp.float32),
            pltpu.VMEM((G, VA), jnp.float32),
            pltpu.VMEM((L,), jnp.float32),
            pltpu.SemaphoreType.DMA,
            pltpu.SemaphoreType.DMA,
            pltpu.SemaphoreType.DMA,
            pltpu.SemaphoreType.DMA,
            pltpu.SemaphoreType.DMA,
        ],
    )
    def k(taba_hbm, tabflat_hbm, idx_hbm, tgt_hbm, logz_hbm, out_hbm,
          part_hbm, idx_v, tgt_v, fidx_v, picked_v, lzp_v, rows0_v, rows1_v,
          acc_v, psem, sin0, sin1, sout0, sout1):
        wid = lax.axis_index("s") * NC + lax.axis_index("c")
        base = pl.multiple_of(wid * RPT, RPT)
        rows = (rows0_v, rows1_v)
        sin = (sin0, sin1)
        sout = (sout0, sout1)
        pltpu.sync_copy(idx_hbm.at[pl.ds(base, RPT)], idx_v)
        pltpu.sync_copy(tgt_hbm.at[pl.ds(base, RPT)], tgt_v)

        def start_in(g, b):
            off = pl.multiple_of(g * G, G)
            pltpu.async_copy(taba_hbm.at[idx_v.at[pl.ds(off, G)]],
                             rows[b], sin[b])

        def wait_in(b):
            pltpu.make_async_copy(taba_hbm.at[idx_v.at[pl.ds(0, G)]],
                                  rows[b], sin[b]).wait()

        def start_out(g, b):
            off = pl.multiple_of(g * G, G)
            pltpu.async_copy(rows[b],
                             out_hbm.at[pl.ds(base + off, G), pl.ds(0, VA)],
                             sout[b])

        def wait_out(b):
            pltpu.make_async_copy(rows[b],
                                  out_hbm.at[pl.ds(base, G), pl.ds(0, VA)],
                                  sout[b]).wait()

        start_in(0, 0)
        start_in(1, 1)

        def mkflat(j, _):
            o = pl.multiple_of(j * L, L)
            fidx_v[pl.ds(o, L)] = idx_v[pl.ds(o, L)] * V + tgt_v[pl.ds(o, L)]
            return 0
        lax.fori_loop(0, RPT // L, mkflat, 0)

        pltpu.async_copy(tabflat_hbm.at[fidx_v], picked_v, psem)
        pltpu.async_copy(logz_hbm.at[idx_v], lzp_v, psem)
        pltpu.make_async_copy(tabflat_hbm.at[fidx_v], picked_v, psem).wait()
        pltpu.make_async_copy(logz_hbm.at[idx_v], lzp_v, psem).wait()

        def accum(j, acc):
            o = pl.multiple_of(j * L, L)
            return acc + (lzp_v[pl.ds(o, L)] - picked_v[pl.ds(o, L)])
        acc = lax.fori_loop(0, RPT // L, accum, jnp.zeros((L,), jnp.float32))
        acc_v[...] = acc
        pltpu.sync_copy(acc_v, part_hbm.at[wid])

        def step(i, _):
            t = i * 2
            for b in range(2):
                g = t + b
                wait_in(b)
                start_out(g, b)

            @pl.when(t + 2 < NCHUNK)
            def _():
                wait_out(0)
                start_in(t + 2, 0)

            @pl.when(t + 3 < NCHUNK)
            def _():
                wait_out(1)
                start_in(t + 3, 1)
            return 0
        lax.fori_loop(0, NCHUNK // 2, step, 0)
        wait_out(0)
        wait_out(1)

    return k(table_a, table_flat, idx_h, tgt_h, logz)


def _oh_tail(idx_ref, tailT_ref):
    ids = idx_ref[0, 0, :]
    ohT = (lax.broadcasted_iota(jnp.int32, (V, GT), 0)
           == ids[None, :]).astype(jnp.float32)
    return jnp.dot(tailT_ref[...], ohT, preferred_element_type=jnp.float32)


def _relayout_a_body(idx_ref, tailT_ref, sc_ref, out_ref):
    out_ref[0:VA, :] = jnp.transpose(sc_ref[...])
    out_ref[VA:V, :] = _oh_tail(idx_ref, tailT_ref)


def _relayout_b_body(idx_ref, tailT_ref, pa_ref, pb_ref, sc_ref, alias_ref,
                     out_ref, loss_ref):
    del alias_ref
    out_ref[0:VA, :] = jnp.transpose(sc_ref[...])
    out_ref[VA:V, :] = _oh_tail(idx_ref, tailT_ref)

    @pl.when(pl.program_id(0) == 0)
    def _():
        loss_ref[0, 0] = (jnp.sum(pa_ref[...]) + jnp.sum(pb_ref[...])) \
            * (1.0 / N)


def _tc_relayout_a(idx3, tailT, sc_a):
    return pl.pallas_call(
        _relayout_a_body,
        grid=(NBH,),
        in_specs=[
            pl.BlockSpec((1, 1, GT), lambda i: (i, 0, 0)),
            pl.BlockSpec((TW, V), lambda i: (0, 0)),
            pl.BlockSpec((GT, VA), lambda i: (i, 0)),
        ],
        out_specs=pl.BlockSpec((V, GT), lambda i: (0, i)),
        out_shape=jax.ShapeDtypeStruct((V, N), jnp.float32),
    )(idx3, tailT, sc_a)


def _tc_relayout_b(idx3, tailT, pa, pb, sc_b, outT):
    return pl.pallas_call(
        _relayout_b_body,
        grid=(NBH,),
        in_specs=[
            pl.BlockSpec((1, 1, GT), lambda i: (i, 0, 0)),
            pl.BlockSpec((TW, V), lambda i: (0, 0)),
            pl.BlockSpec((NW, L), lambda i: (0, 0)),
            pl.BlockSpec((NW, L), lambda i: (0, 0)),
            pl.BlockSpec((GT, VA), lambda i: (i, 0)),
            pl.BlockSpec(memory_space=pl.ANY),
        ],
        out_specs=[
            pl.BlockSpec((V, GT), lambda i: (0, i + NBH)),
            pl.BlockSpec((1, 1), lambda i: (0, 0),
                         memory_space=pltpu.SMEM),
        ],
        out_shape=[
            jax.ShapeDtypeStruct((V, N), jnp.float32),
            jax.ShapeDtypeStruct((1, 1), jnp.float32),
        ],
        input_output_aliases={5: 0},
    )(idx3, tailT, pa, pb, sc_b, outT)


def kernel(idx, target, table):
    idx_flat = idx.reshape(N)
    tgt_flat = target.reshape(N)
    logz = _tc_logz(table).reshape(V)
    table_a = table[:, :VA]
    table_flat = table.reshape(V * V)
    tailT = table[:, TSTART:].T
    idx_a, idx_b = idx_flat[:NH], idx_flat[NH:]
    tgt_a, tgt_b = tgt_flat[:NH], tgt_flat[NH:]
    sc_a, pa = _sc_gather(table_a, table_flat, idx_a, tgt_a, logz)
    sc_b, pb = _sc_gather(table_a, table_flat, idx_b, tgt_b, logz)
    outT_a = _tc_relayout_a(idx_a.reshape(NBH, 1, GT), tailT, sc_a)
    outT, loss = _tc_relayout_b(idx_b.reshape(NBH, 1, GT), tailT, pa, pb,
                                sc_b, outT_a)
    return (jnp.transpose(outT), loss[0, 0])


# speedup vs baseline: 2.3598x; 1.0023x over previous
import functools

import jax
import jax.numpy as jnp
from jax import lax
from jax.experimental import pallas as pl
from jax.experimental.pallas import tpu as pltpu
from jax.experimental.pallas import tpu_sc as plsc

V = 1000
N = 51200
NH = N // 2
VA = 896
NC, NS, L = 2, 16, 16
NW = NC * NS
RPT = NH // NW
G = 16
NCHUNK = RPT // G
GT = 1280
NBH = NH // GT
TW = V - VA
TSTART = VA


def _logz_body(tab_ref, out_ref):
    x = tab_ref[:]
    m = jnp.max(x, axis=1, keepdims=True)
    s = jnp.sum(jnp.exp(x - m), axis=1, keepdims=True)
    out_ref[:] = m + jnp.log(s)


def _tc_logz(table):
    return pl.pallas_call(
        _logz_body,
        out_shape=jax.ShapeDtypeStruct((V, 1), jnp.float32),
    )(table)


def _sc_gather(table_a, table_flat, idx_h, tgt_h, logz):
    mesh = plsc.VectorSubcoreMesh(core_axis_name="c", subcore_axis_name="s")

    @functools.partial(
        pl.kernel,
        out_type=(
            jax.ShapeDtypeStruct((NH, VA), jnp.float32),
            jax.ShapeDtypeStruct((NW, L), jnp.float32),
        ),
        mesh=mesh,
        scratch_types=[
            pltpu.VMEM((RPT,), jnp.int32),
            pltpu.VMEM((RPT,), jnp.int32),
            pltpu.VMEM((RPT,), jnp.int32),
            pltpu.VMEM((RPT,), jnp.float32),
            pltpu.VMEM((RPT,), jnp.float32),
            pltpu.VMEM((G, VA), jnp.float32),
            pltpu.VMEM((G, VA), jnp.float32),
            pltpu.VMEM((L,), jnp.float32),
            pltpu.SemaphoreType.DMA,
            pltpu.SemaphoreType.DMA,
            pltpu.SemaphoreType.DMA,
            pltpu.SemaphoreType.DMA,
            pltpu.SemaphoreType.DMA,
        ],
    )
    def k(taba_hbm, tabflat_hbm, idx_hbm, tgt_hbm, logz_hbm, out_hbm,
          part_hbm, idx_v, tgt_v, fidx_v, picked_v, lzp_v, rows0_v, rows1_v,
          acc_v, psem, sin0, sin1, sout0, sout1):
        wid = lax.axis_index("s") * NC + lax.axis_index("c")
        base = pl.multiple_of(wid * RPT, RPT)
        rows = (rows0_v, rows1_v)
        sin = (sin0, sin1)
        sout = (sout0, sout1)
        pltpu.sync_copy(idx_hbm.at[pl.ds(base, RPT)], idx_v)
        pltpu.sync_copy(tgt_hbm.at[pl.ds(base, RPT)], tgt_v)

        def start_in(g, b):
            off = pl.multiple_of(g * G, G)
            pltpu.async_copy(taba_hbm.at[idx_v.at[pl.ds(off, G)]],
                             rows[b], sin[b])

        def wait_in(b):
            pltpu.make_async_copy(taba_hbm.at[idx_v.at[pl.ds(0, G)]],
                                  rows[b], sin[b]).wait()

        def start_out(g, b):
            off = pl.multiple_of(g * G, G)
            pltpu.async_copy(rows[b],
                             out_hbm.at[pl.ds(base + off, G), pl.ds(0, VA)],
                             sout[b])

        def wait_out(b):
            pltpu.make_async_copy(rows[b],
                                  out_hbm.at[pl.ds(base, G), pl.ds(0, VA)],
                                  sout[b]).wait()

        start_in(0, 0)
        start_in(1, 1)

        def mkflat(j, _):
            o = pl.multiple_of(j * L, L)
            fidx_v[pl.ds(o, L)] = idx_v[pl.ds(o, L)] * V + tgt_v[pl.ds(o, L)]
            return 0
        lax.fori_loop(0, RPT // L, mkflat, 0)

        pltpu.async_copy(tabflat_hbm.at[fidx_v], picked_v, psem)
        pltpu.async_copy(logz_hbm.at[idx_v], lzp_v, psem)
        pltpu.make_async_copy(tabflat_hbm.at[fidx_v], picked_v, psem).wait()
        pltpu.make_async_copy(logz_hbm.at[idx_v], lzp_v, psem).wait()

        def accum(j, acc):
            o = pl.multiple_of(j * L, L)
            return acc + (lzp_v[pl.ds(o, L)] - picked_v[pl.ds(o, L)])
        acc = lax.fori_loop(0, RPT // L, accum, jnp.zeros((L,), jnp.float32))
        acc_v[...] = acc
        pltpu.sync_copy(acc_v, part_hbm.at[wid])

        def step(i, _):
            t = i * 2
            for b in range(2):
                g = t + b
                wait_in(b)
                start_out(g, b)

            @pl.when(t + 2 < NCHUNK)
            def _():
                wait_out(0)
                start_in(t + 2, 0)

            @pl.when(t + 3 < NCHUNK)
            def _():
                wait_out(1)
                start_in(t + 3, 1)
            return 0
        lax.fori_loop(0, NCHUNK // 2, step, 0)
        wait_out(0)
        wait_out(1)

    return k(table_a, table_flat, idx_h, tgt_h, logz)


def _oh_tail(idx_ref, tailT_ref):
    ids = idx_ref[0, 0, :]
    ohT = (lax.broadcasted_iota(jnp.int32, (V, GT), 0)
           == ids[None, :]).astype(jnp.float32)
    return jnp.dot(tailT_ref[...], ohT, preferred_element_type=jnp.float32)


def _relayout_a_body(idx_ref, tailT_ref, sc_ref, out_ref):
    out_ref[0:VA, :] = jnp.transpose(sc_ref[...])
    out_ref[VA:V, :] = _oh_tail(idx_ref, tailT_ref)


def _relayout_b_body(idx_ref, tailT_ref, pa_ref, pb_ref, sc_ref, alias_ref,
                     out_ref, loss_ref):
    del alias_ref
    out_ref[0:VA, :] = jnp.transpose(sc_ref[...])
    out_ref[VA:V, :] = _oh_tail(idx_ref, tailT_ref)

    @pl.when(pl.program_id(0) == 0)
    def _():
        loss_ref[0, 0] = (jnp.sum(pa_ref[...]) + jnp.sum(pb_ref[...])) \
            * (1.0 / N)


def _tc_relayout_a(idx3, tailT, sc_a):
    return pl.pallas_call(
        _relayout_a_body,
        grid=(NBH,),
        in_specs=[
            pl.BlockSpec((1, 1, GT), lambda i: (i, 0, 0)),
            pl.BlockSpec((TW, V), lambda i: (0, 0)),
            pl.BlockSpec((GT, VA), lambda i: (i, 0)),
        ],
        out_specs=pl.BlockSpec((V, GT), lambda i: (0, i)),
        out_shape=jax.ShapeDtypeStruct((V, N), jnp.float32),
    )(idx3, tailT, sc_a)


def _tc_relayout_b(idx3, tailT, pa, pb, sc_b, outT):
    return pl.pallas_call(
        _relayout_b_body,
        grid=(NBH,),
        in_specs=[
            pl.BlockSpec((1, 1, GT), lambda i: (i, 0, 0)),
            pl.BlockSpec((TW, V), lambda i: (0, 0)),
            pl.BlockSpec((NW, L), lambda i: (0, 0)),
            pl.BlockSpec((NW, L), lambda i: (0, 0)),
            pl.BlockSpec((GT, VA), lambda i: (i, 0)),
            pl.BlockSpec(memory_space=pl.ANY),
        ],
        out_specs=[
            pl.BlockSpec((V, GT), lambda i: (0, i + NBH)),
            pl.BlockSpec((1, 1), lambda i: (0, 0),
                         memory_space=pltpu.SMEM),
        ],
        out_shape=[
            jax.ShapeDtypeStruct((V, N), jnp.float32),
            jax.ShapeDtypeStruct((1, 1), jnp.float32),
        ],
        input_output_aliases={5: 0},
    )(idx3, tailT, pa, pb, sc_b, outT)


def kernel(idx, target, table):
    idx_flat = idx.reshape(N)
    tgt_flat = target.reshape(N)
    logz = _tc_logz(table).reshape(V)
    table_a = table[:, :VA]
    table_flat = table.reshape(V * V)
    tailT = table[:, TSTART:].T
    idx_a, idx_b = idx_flat[:NH], idx_flat[NH:]
    tgt_a, tgt_b = tgt_flat[:NH], tgt_flat[NH:]
    sc_a, pa = _sc_gather(table_a, table_flat, idx_a, tgt_a, logz)
    sc_b, pb = _sc_gather(table_a, table_flat, idx_b, tgt_b, logz)
    outT_a = _tc_relayout_a(idx_a.reshape(NBH, 1, GT), tailT, sc_a)
    outT, loss = _tc_relayout_b(idx_b.reshape(NBH, 1, GT), tailT, pa, pb,
                                sc_b, outT_a)
    return (jnp.transpose(outT), loss[0, 0])


# speedup vs baseline: 2.3805x; 1.0088x over previous
import functools

import jax
import jax.numpy as jnp
from jax import lax
from jax.experimental import pallas as pl
from jax.experimental.pallas import tpu as pltpu
from jax.experimental.pallas import tpu_sc as plsc

V = 1000
N = 51200
NH = N // 2
VA = 896
NC, NS, L = 2, 16, 16
NW = NC * NS
RPT = NH // NW
G = 40
NCHUNK = RPT // G
GT = 1280
NBH = NH // GT
TW = V - VA
TSTART = VA


def _logz_body(tab_ref, out_ref):
    x = tab_ref[:]
    m = jnp.max(x, axis=1, keepdims=True)
    s = jnp.sum(jnp.exp(x - m), axis=1, keepdims=True)
    out_ref[:] = m + jnp.log(s)


def _tc_logz(table):
    return pl.pallas_call(
        _logz_body,
        out_shape=jax.ShapeDtypeStruct((V, 1), jnp.float32),
    )(table)


def _sc_gather(table_a, table_flat, idx_h, tgt_h, logz):
    mesh = plsc.VectorSubcoreMesh(core_axis_name="c", subcore_axis_name="s")

    @functools.partial(
        pl.kernel,
        out_type=(
            jax.ShapeDtypeStruct((NH, VA), jnp.float32),
            jax.ShapeDtypeStruct((NW, L), jnp.float32),
        ),
        mesh=mesh,
        scratch_types=[
            pltpu.VMEM((RPT,), jnp.int32),
            pltpu.VMEM((RPT,), jnp.int32),
            pltpu.VMEM((RPT,), jnp.int32),
            pltpu.VMEM((RPT,), jnp.float32),
            pltpu.VMEM((RPT,), jnp.float32),
            pltpu.VMEM((G, VA), jnp.float32),
            pltpu.VMEM((G, VA), jnp.float32),
            pltpu.VMEM((L,), jnp.float32),
            pltpu.SemaphoreType.DMA,
            pltpu.SemaphoreType.DMA,
            pltpu.SemaphoreType.DMA,
            pltpu.SemaphoreType.DMA,
            pltpu.SemaphoreType.DMA,
        ],
    )
    def k(taba_hbm, tabflat_hbm, idx_hbm, tgt_hbm, logz_hbm, out_hbm,
          part_hbm, idx_v, tgt_v, fidx_v, picked_v, lzp_v, rows0_v, rows1_v,
          acc_v, psem, sin0, sin1, sout0, sout1):
        wid = lax.axis_index("s") * NC + lax.axis_index("c")
        base = pl.multiple_of(wid * RPT, RPT)
        rows = (rows0_v, rows1_v)
        sin = (sin0, sin1)
        sout = (sout0, sout1)
        pltpu.sync_copy(idx_hbm.at[pl.ds(base, RPT)], idx_v)
        pltpu.sync_copy(tgt_hbm.at[pl.ds(base, RPT)], tgt_v)

        def start_in(g, b):
            off = pl.multiple_of(g * G, G)
            pltpu.async_copy(taba_hbm.at[idx_v.at[pl.ds(off, G)]],
                             rows[b], sin[b])

        def wait_in(b):
            pltpu.make_async_copy(taba_hbm.at[idx_v.at[pl.ds(0, G)]],
                                  rows[b], sin[b]).wait()

        def start_out(g, b):
            off = pl.multiple_of(g * G, G)
            pltpu.async_copy(rows[b],
                             out_hbm.at[pl.ds(base + off, G), pl.ds(0, VA)],
                             sout[b])

        def wait_out(b):
            pltpu.make_async_copy(rows[b],
                                  out_hbm.at[pl.ds(base, G), pl.ds(0, VA)],
                                  sout[b]).wait()

        start_in(0, 0)
        start_in(1, 1)

        def mkflat(j, _):
            o = pl.multiple_of(j * L, L)
            fidx_v[pl.ds(o, L)] = idx_v[pl.ds(o, L)] * V + tgt_v[pl.ds(o, L)]
            return 0
        lax.fori_loop(0, RPT // L, mkflat, 0)

        pltpu.async_copy(tabflat_hbm.at[fidx_v], picked_v, psem)
        pltpu.async_copy(logz_hbm.at[idx_v], lzp_v, psem)
        pltpu.make_async_copy(tabflat_hbm.at[fidx_v], picked_v, psem).wait()
        pltpu.make_async_copy(logz_hbm.at[idx_v], lzp_v, psem).wait()

        def accum(j, acc):
            o = pl.multiple_of(j * L, L)
            return acc + (lzp_v[pl.ds(o, L)] - picked_v[pl.ds(o, L)])
        acc = lax.fori_loop(0, RPT // L, accum, jnp.zeros((L,), jnp.float32))
        acc_v[...] = acc
        pltpu.sync_copy(acc_v, part_hbm.at[wid])

        def step(i, _):
            t = i * 2
            for b in range(2):
                g = t + b
                wait_in(b)
                start_out(g, b)

            @pl.when(t + 2 < NCHUNK)
            def _():
                wait_out(0)
                start_in(t + 2, 0)

            @pl.when(t + 3 < NCHUNK)
            def _():
                wait_out(1)
                start_in(t + 3, 1)
            return 0
        lax.fori_loop(0, NCHUNK // 2, step, 0)
        wait_out(0)
        wait_out(1)

    return k(table_a, table_flat, idx_h, tgt_h, logz)


def _oh_tail(idx_ref, tailT_ref):
    ids = idx_ref[0, 0, :]
    ohT = (lax.broadcasted_iota(jnp.int32, (V, GT), 0)
           == ids[None, :]).astype(jnp.float32)
    return jnp.dot(tailT_ref[...], ohT, preferred_element_type=jnp.float32)


def _relayout_a_body(idx_ref, tailT_ref, sc_ref, out_ref):
    out_ref[0:VA, :] = jnp.transpose(sc_ref[...])
    out_ref[VA:V, :] = _oh_tail(idx_ref, tailT_ref)


def _relayout_b_body(idx_ref, tailT_ref, pa_ref, pb_ref, sc_ref, alias_ref,
                     out_ref, loss_ref):
    del alias_ref
    out_ref[0:VA, :] = jnp.transpose(sc_ref[...])
    out_ref[VA:V, :] = _oh_tail(idx_ref, tailT_ref)

    @pl.when(pl.program_id(0) == 0)
    def _():
        loss_ref[0, 0] = (jnp.sum(pa_ref[...]) + jnp.sum(pb_ref[...])) \
            * (1.0 / N)


def _tc_relayout_a(idx3, tailT, sc_a):
    return pl.pallas_call(
        _relayout_a_body,
        grid=(NBH,),
        in_specs=[
            pl.BlockSpec((1, 1, GT), lambda i: (i, 0, 0)),
            pl.BlockSpec((TW, V), lambda i: (0, 0)),
            pl.BlockSpec((GT, VA), lambda i: (i, 0)),
        ],
        out_specs=pl.BlockSpec((V, GT), lambda i: (0, i)),
        out_shape=jax.ShapeDtypeStruct((V, N), jnp.float32),
    )(idx3, tailT, sc_a)


def _tc_relayout_b(idx3, tailT, pa, pb, sc_b, outT):
    return pl.pallas_call(
        _relayout_b_body,
        grid=(NBH,),
        in_specs=[
            pl.BlockSpec((1, 1, GT), lambda i: (i, 0, 0)),
            pl.BlockSpec((TW, V), lambda i: (0, 0)),
            pl.BlockSpec((NW, L), lambda i: (0, 0)),
            pl.BlockSpec((NW, L), lambda i: (0, 0)),
            pl.BlockSpec((GT, VA), lambda i: (i, 0)),
            pl.BlockSpec(memory_space=pl.ANY),
        ],
        out_specs=[
            pl.BlockSpec((V, GT), lambda i: (0, i + NBH)),
            pl.BlockSpec((1, 1), lambda i: (0, 0),
                         memory_space=pltpu.SMEM),
        ],
        out_shape=[
            jax.ShapeDtypeStruct((V, N), jnp.float32),
            jax.ShapeDtypeStruct((1, 1), jnp.float32),
        ],
        input_output_aliases={5: 0},
    )(idx3, tailT, pa, pb, sc_b, outT)


def kernel(idx, target, table):
    idx_flat = idx.reshape(N)
    tgt_flat = target.reshape(N)
    logz = _tc_logz(table).reshape(V)
    table_a = table[:, :VA]
    table_flat = table.reshape(V * V)
    tailT = table[:, TSTART:].T
    idx_a, idx_b = idx_flat[:NH], idx_flat[NH:]
    tgt_a, tgt_b = tgt_flat[:NH], tgt_flat[NH:]
    sc_a, pa = _sc_gather(table_a, table_flat, idx_a, tgt_a, logz)
    sc_b, pb = _sc_gather(table_a, table_flat, idx_b, tgt_b, logz)
    outT_a = _tc_relayout_a(idx_a.reshape(NBH, 1, GT), tailT, sc_a)
    outT, loss = _tc_relayout_b(idx_b.reshape(NBH, 1, GT), tailT, pa, pb,
                                sc_b, outT_a)
    return (jnp.transpose(outT), loss[0, 0])
